# baseline (speedup 1.0000x reference)
import functools

import jax
import jax.numpy as jnp
from jax.experimental import pallas as pl
from jax.experimental.pallas import tpu as pltpu
from jax.experimental.pallas import tpu_sc as plsc

E = 64
H = 768
I = 1536
T = 2048
TOP_K = 2
TK = T * TOP_K

TB = 256
M = 64
NPT = TK // M + E
PAD = NPT * M

NW = 32
GCH = 64


def _router_body(x_ref, gw_ref, rw_ref, se_ref):
    xb = x_ref[...]
    gw = gw_ref[...]
    logits = jax.lax.dot_general(xb, gw, (((1,), (1,)), ((), ())),
                                 preferred_element_type=jnp.float32)
    m = jnp.max(logits, axis=1, keepdims=True)
    ex = jnp.exp(logits - m)
    probs = ex / jnp.sum(ex, axis=1, keepdims=True)

    ids = jax.lax.broadcasted_iota(jnp.int32, (TB, E), 1)
    i1 = jnp.argmax(probs, axis=1).astype(jnp.int32)[:, None]
    w1v = jnp.max(probs, axis=1, keepdims=True)
    probs2 = jnp.where(ids == i1, -jnp.inf, probs)
    i2 = jnp.argmax(probs2, axis=1).astype(jnp.int32)[:, None]
    w2v = jnp.max(probs2, axis=1, keepdims=True)
    rw_ref[...] = jnp.concatenate([w1v, w2v], axis=1)
    se_ref[...] = jnp.concatenate([i1, i2], axis=1)


def _sc_gather(n_rows):
    per_w = n_rows // NW
    n_ch = per_w // GCH
    mesh = plsc.VectorSubcoreMesh(core_axis_name="c", subcore_axis_name="s")

    def body(idx_hbm, tab_hbm, out_hbm, idx_v, rows_v, sem):
        wid = jax.lax.axis_index("s") * 2 + jax.lax.axis_index("c")
        base = wid * per_w
        for c in range(n_ch):
            off = base + c * GCH
            pltpu.sync_copy(idx_hbm.at[pl.ds(off, GCH)], idx_v)
            pltpu.async_copy(tab_hbm.at[idx_v], rows_v, sem).wait()
            pltpu.sync_copy(rows_v, out_hbm.at[pl.ds(off, GCH)])

    return pl.kernel(
        body, mesh=mesh,
        out_type=jax.ShapeDtypeStruct((n_rows, H), jnp.float32),
        scratch_types=[
            pltpu.VMEM((GCH,), jnp.int32),
            pltpu.VMEM((GCH, H), jnp.float32),
            pltpu.SemaphoreType.DMA,
        ],
    )


def _expert_body(e_of_ref, xs_ref, sw_ref, w1_ref, w3_ref, w2_ref, ys_ref):
    xb = xs_ref[...].astype(jnp.bfloat16)
    swb = sw_ref[0]
    w1b = w1_ref[0].astype(jnp.bfloat16)
    w3b = w3_ref[0].astype(jnp.bfloat16)
    w2b = w2_ref[0].astype(jnp.bfloat16)

    a = jax.lax.dot_general(xb, w1b, (((1,), (1,)), ((), ())),
                            preferred_element_type=jnp.float32)
    b = jax.lax.dot_general(xb, w3b, (((1,), (1,)), ((), ())),
                            preferred_element_type=jnp.float32)
    h = (a * jax.nn.sigmoid(a) * b).astype(jnp.bfloat16)
    y = jax.lax.dot_general(h, w2b, (((1,), (1,)), ((), ())),
                            preferred_element_type=jnp.float32)
    ys_ref[...] = y * swb


def _pair_add_body(ysg_ref, out_ref):
    out_ref[...] = ysg_ref[:, 0, :] + ysg_ref[:, 1, :]


@jax.jit
def kernel(x, gate_w, w1, w2, w3):
    rw, se = pl.pallas_call(
        _router_body,
        grid=(T // TB,),
        in_specs=[
            pl.BlockSpec((TB, H), lambda t: (t, 0)),
            pl.BlockSpec((E, H), lambda t: (0, 0)),
        ],
        out_specs=[
            pl.BlockSpec((TB, TOP_K), lambda t: (t, 0)),
            pl.BlockSpec((TB, TOP_K), lambda t: (t, 0)),
        ],
        out_shape=[
            jax.ShapeDtypeStruct((T, TOP_K), jnp.float32),
            jax.ShapeDtypeStruct((T, TOP_K), jnp.int32),
        ],
    )(x, gate_w)

    f = se.reshape(-1)
    rwf = rw.reshape(-1)
    counts = jnp.bincount(f, length=E)
    tiles = (counts + M - 1) // M
    csum_tiles = jnp.cumsum(tiles)
    tile_start = csum_tiles - tiles
    e_of = jnp.searchsorted(csum_tiles, jnp.arange(NPT), side="right")
    e_of = jnp.minimum(e_of, E - 1).astype(jnp.int32)

    order = jnp.argsort(f, stable=True)
    fs = f[order]
    grp_start = (jnp.cumsum(counts) - counts)[fs]
    rank = jnp.arange(TK, dtype=jnp.int32) - grp_start.astype(jnp.int32)
    pos = tile_start[fs].astype(jnp.int32) * M + rank

    st = jnp.zeros((PAD,), jnp.int32).at[pos].set(
        (order // TOP_K).astype(jnp.int32))
    sw = jnp.zeros((PAD,), jnp.float32).at[pos].set(rwf[order])
    sw3 = sw.reshape(NPT, M, 1)
    ps = jnp.zeros((TK,), jnp.int32).at[order].set(pos)

    xs = _sc_gather(PAD)(st, x)

    ys = pl.pallas_call(
        _expert_body,
        grid_spec=pltpu.PrefetchScalarGridSpec(
            num_scalar_prefetch=1,
            grid=(NPT,),
            in_specs=[
                pl.BlockSpec((M, H), lambda g, eo: (g, 0)),
                pl.BlockSpec((1, M, 1), lambda g, eo: (g, 0, 0)),
                pl.BlockSpec((1, I, H), lambda g, eo: (eo[g], 0, 0)),
                pl.BlockSpec((1, I, H), lambda g, eo: (eo[g], 0, 0)),
                pl.BlockSpec((1, H, I), lambda g, eo: (eo[g], 0, 0)),
            ],
            out_specs=pl.BlockSpec((M, H), lambda g, eo: (g, 0)),
        ),
        out_shape=jax.ShapeDtypeStruct((PAD, H), jnp.float32),
    )(e_of, xs, sw3, w1, w3, w2)

    ysg = _sc_gather(TK)(ps, ys).reshape(T, TOP_K, H)

    out = pl.pallas_call(
        _pair_add_body,
        grid=(T // TB,),
        in_specs=[pl.BlockSpec((TB, TOP_K, H), lambda t: (t, 0, 0))],
        out_specs=pl.BlockSpec((TB, H), lambda t: (t, 0)),
        out_shape=jax.ShapeDtypeStruct((T, H), jnp.float32),
    )(ysg)
    return out


# speedup vs baseline: 1.5340x; 1.5340x over previous
import jax
import jax.numpy as jnp
from jax.experimental import pallas as pl
from jax.experimental.pallas import tpu as pltpu
from jax.experimental.pallas import tpu_sc as plsc

E = 64
H = 768
I = 1536
T = 2048
TOP_K = 2
TK = T * TOP_K

TB = 256
NCH = T // TB
M = 64
NPT = TK // M + E
PAD = NPT * M

NW = 32
TPW = T // NW
CCH = 32


def _router_body(x_ref, gw_ref, rw_ref, se_ref):
    xb = x_ref[...]
    gw = gw_ref[...]
    logits = jax.lax.dot_general(xb, gw, (((1,), (1,)), ((), ())),
                                 preferred_element_type=jnp.float32)
    m = jnp.max(logits, axis=1, keepdims=True)
    ex = jnp.exp(logits - m)
    probs = ex / jnp.sum(ex, axis=1, keepdims=True)

    ids = jax.lax.broadcasted_iota(jnp.int32, (TB, E), 1)
    i1 = jnp.argmax(probs, axis=1).astype(jnp.int32)[:, None]
    w1v = jnp.max(probs, axis=1, keepdims=True)
    probs2 = jnp.where(ids == i1, -jnp.inf, probs)
    i2 = jnp.argmax(probs2, axis=1).astype(jnp.int32)[:, None]
    w2v = jnp.max(probs2, axis=1, keepdims=True)
    rw_ref[...] = jnp.concatenate([w1v, w2v], axis=1)
    se_ref[...] = jnp.concatenate([i1, i2], axis=1)


def _pos_body(se_ref, ps_ref, eof_ref, run_ref, rs_ref):
    g = pl.program_id(0)
    seb = se_ref[...]
    ids = jax.lax.broadcasted_iota(jnp.int32, (TB, E), 1)
    oh0 = (ids == seb[:, 0:1]).astype(jnp.float32)
    oh1 = (ids == seb[:, 1:2]).astype(jnp.float32)
    ocnt = oh0 + oh1

    @pl.when(g == 0)
    def _():
        run_ref[...] = jnp.zeros((1, E), jnp.float32)

    @pl.when(g < NCH)
    def _():
        run_ref[...] += jnp.sum(ocnt, axis=0, keepdims=True)

    @pl.when(g == NCH)
    def _():
        totals = run_ref[...]
        tiles = jnp.floor((totals + (M - 1)) * (1.0 / M))
        ei = jax.lax.broadcasted_iota(jnp.int32, (E, E), 0)
        ej = jax.lax.broadcasted_iota(jnp.int32, (E, E), 1)
        lstrict_e = (ei < ej).astype(jnp.float32)
        ts_tiles = jax.lax.dot_general(
            tiles, lstrict_e, (((1,), (0,)), ((), ())),
            preferred_element_type=jnp.float32)
        rs_ref[...] = ts_tiles * float(M)
        run_ref[...] = jnp.zeros((1, E), jnp.float32)
        cum_incl = ts_tiles + tiles
        gi = jax.lax.broadcasted_iota(jnp.int32, (NPT, E), 0).astype(jnp.float32)
        eof = jnp.sum((gi >= cum_incl).astype(jnp.float32), axis=1,
                      keepdims=True)
        eof_ref[...] = jnp.minimum(eof, float(E - 1)).astype(jnp.int32)

    @pl.when(g >= NCH)
    def _():
        ri = jax.lax.broadcasted_iota(jnp.int32, (TB, TB), 0)
        rj = jax.lax.broadcasted_iota(jnp.int32, (TB, TB), 1)
        lstrict = (rj < ri).astype(jnp.float32)
        crow = jax.lax.dot_general(
            lstrict, ocnt, (((1,), (0,)), ((), ())),
            preferred_element_type=jnp.float32)
        base = rs_ref[...] + run_ref[...] + crow
        p0 = jnp.sum(oh0 * base, axis=1, keepdims=True)
        p1 = jnp.sum(oh1 * (base + oh0), axis=1, keepdims=True)
        ps_ref[...] = jnp.concatenate([p0, p1], axis=1).astype(jnp.int32)
        run_ref[...] += jnp.sum(ocnt, axis=0, keepdims=True)


def _make_sc_dispatch():
    mesh = plsc.VectorSubcoreMesh(core_axis_name="c", subcore_axis_name="s")

    def body(x_hbm, ps0_hbm, ps1_hbm, xs_hbm, idx_v, rows_v, sem):
        wid = jax.lax.axis_index("s") * 2 + jax.lax.axis_index("c")
        for c in range(TPW // CCH):
            tbase = wid * TPW + c * CCH
            pltpu.sync_copy(x_hbm.at[pl.ds(tbase, CCH)], rows_v)
            pltpu.sync_copy(ps0_hbm.at[wid, c], idx_v)
            pltpu.async_copy(rows_v, xs_hbm.at[idx_v], sem).wait()
            pltpu.sync_copy(ps1_hbm.at[wid, c], idx_v)
            pltpu.async_copy(rows_v, xs_hbm.at[idx_v], sem).wait()

    return pl.kernel(
        body, mesh=mesh,
        out_type=jax.ShapeDtypeStruct((PAD, H), jnp.float32),
        scratch_types=[
            pltpu.VMEM((CCH,), jnp.int32),
            pltpu.VMEM((CCH, H), jnp.float32),
            pltpu.SemaphoreType.DMA,
        ],
    )


def _make_sc_combine():
    mesh = plsc.VectorSubcoreMesh(core_axis_name="c", subcore_axis_name="s")
    per_w = TK // NW
    n_ch = per_w // (2 * CCH)

    def body(ps_hbm, ys_hbm, ysg_hbm, idx_v, rows_v, sem):
        wid = jax.lax.axis_index("s") * 2 + jax.lax.axis_index("c")
        base = wid * per_w
        for c in range(n_ch):
            off = base + c * (2 * CCH)
            pltpu.sync_copy(ps_hbm.at[pl.ds(off, 2 * CCH)], idx_v)
            pltpu.async_copy(ys_hbm.at[idx_v], rows_v, sem).wait()
            pltpu.sync_copy(rows_v, ysg_hbm.at[pl.ds(off, 2 * CCH)])

    return pl.kernel(
        body, mesh=mesh,
        out_type=jax.ShapeDtypeStruct((TK, H), jnp.float32),
        scratch_types=[
            pltpu.VMEM((2 * CCH,), jnp.int32),
            pltpu.VMEM((2 * CCH, H), jnp.float32),
            pltpu.SemaphoreType.DMA,
        ],
    )


def _sc_dispatch(x, ps0, ps1):
    return _make_sc_dispatch()(x, ps0, ps1)


def _sc_combine(ps, ys):
    return _make_sc_combine()(ps, ys)


def _expert_body(e_of_ref, xs_ref, w1_ref, w3_ref, w2_ref, ys_ref):
    xb = xs_ref[...].astype(jnp.bfloat16)
    w1b = w1_ref[0].astype(jnp.bfloat16)
    w3b = w3_ref[0].astype(jnp.bfloat16)
    w2b = w2_ref[0].astype(jnp.bfloat16)

    a = jax.lax.dot_general(xb, w1b, (((1,), (1,)), ((), ())),
                            preferred_element_type=jnp.float32)
    b = jax.lax.dot_general(xb, w3b, (((1,), (1,)), ((), ())),
                            preferred_element_type=jnp.float32)
    h = (a * jax.nn.sigmoid(a) * b).astype(jnp.bfloat16)
    ys_ref[...] = jax.lax.dot_general(h, w2b, (((1,), (1,)), ((), ())),
                                      preferred_element_type=jnp.float32)


def _combine_body(ysg_ref, rw_ref, out_ref):
    rwb = rw_ref[...]
    out_ref[...] = (rwb[:, 0:1] * ysg_ref[:, 0, :]
                    + rwb[:, 1:2] * ysg_ref[:, 1, :])


@jax.jit
def kernel(x, gate_w, w1, w2, w3):
    rw, se = pl.pallas_call(
        _router_body,
        grid=(NCH,),
        in_specs=[
            pl.BlockSpec((TB, H), lambda t: (t, 0)),
            pl.BlockSpec((E, H), lambda t: (0, 0)),
        ],
        out_specs=[
            pl.BlockSpec((TB, TOP_K), lambda t: (t, 0)),
            pl.BlockSpec((TB, TOP_K), lambda t: (t, 0)),
        ],
        out_shape=[
            jax.ShapeDtypeStruct((T, TOP_K), jnp.float32),
            jax.ShapeDtypeStruct((T, TOP_K), jnp.int32),
        ],
    )(x, gate_w)

    ps, e_of = pl.pallas_call(
        _pos_body,
        grid=(2 * NCH,),
        in_specs=[
            pl.BlockSpec((TB, TOP_K),
                         lambda g: (jnp.where(g < NCH, g, g - NCH), 0)),
        ],
        out_specs=[
            pl.BlockSpec((TB, TOP_K),
                         lambda g: (jnp.where(g < NCH, 0, g - NCH), 0)),
            pl.BlockSpec((NPT, 1), lambda g: (0, 0)),
        ],
        out_shape=[
            jax.ShapeDtypeStruct((T, TOP_K), jnp.int32),
            jax.ShapeDtypeStruct((NPT, 1), jnp.int32),
        ],
        scratch_shapes=[
            pltpu.VMEM((1, E), jnp.float32),
            pltpu.VMEM((1, E), jnp.float32),
        ],
    )(se)
    e_of = e_of.reshape(NPT)

    ps0 = ps[:, 0].reshape(NW, TPW // CCH, CCH)
    ps1 = ps[:, 1].reshape(NW, TPW // CCH, CCH)

    xs = _sc_dispatch(x, ps0, ps1)

    ys = pl.pallas_call(
        _expert_body,
        grid_spec=pltpu.PrefetchScalarGridSpec(
            num_scalar_prefetch=1,
            grid=(NPT,),
            in_specs=[
                pl.BlockSpec((M, H), lambda g, eo: (g, 0)),
                pl.BlockSpec((1, I, H), lambda g, eo: (eo[g], 0, 0)),
                pl.BlockSpec((1, I, H), lambda g, eo: (eo[g], 0, 0)),
                pl.BlockSpec((1, H, I), lambda g, eo: (eo[g], 0, 0)),
            ],
            out_specs=pl.BlockSpec((M, H), lambda g, eo: (g, 0)),
        ),
        out_shape=jax.ShapeDtypeStruct((PAD, H), jnp.float32),
    )(e_of, xs, w1, w3, w2)

    ysg = _sc_combine(ps.reshape(TK), ys).reshape(T, TOP_K, H)

    out = pl.pallas_call(
        _combine_body,
        grid=(NCH,),
        in_specs=[
            pl.BlockSpec((TB, TOP_K, H), lambda t: (t, 0, 0)),
            pl.BlockSpec((TB, TOP_K), lambda t: (t, 0)),
        ],
        out_specs=pl.BlockSpec((TB, H), lambda t: (t, 0)),
        out_shape=jax.ShapeDtypeStruct((T, H), jnp.float32),
    )(ysg, rw)
    return out


# speedup vs baseline: 1.7057x; 1.1119x over previous
import jax
import jax.numpy as jnp
from jax.experimental import pallas as pl
from jax.experimental.pallas import tpu as pltpu
from jax.experimental.pallas import tpu_sc as plsc

E = 64
H = 768
I = 1536
T = 2048
TOP_K = 2
TK = T * TOP_K

TB = 256
NCH = T // TB
M = 64
NPT = TK // M + E
PAD = NPT * M

NW = 32
TPW = T // NW
CCH = 32


def _router_body(x_ref, gw_ref, rw_ref, se_ref):
    xb = x_ref[...]
    gw = gw_ref[...]
    logits = jax.lax.dot_general(xb, gw, (((1,), (1,)), ((), ())),
                                 preferred_element_type=jnp.float32)
    m = jnp.max(logits, axis=1, keepdims=True)
    ex = jnp.exp(logits - m)
    probs = ex / jnp.sum(ex, axis=1, keepdims=True)

    ids = jax.lax.broadcasted_iota(jnp.int32, (TB, E), 1)
    i1 = jnp.argmax(probs, axis=1).astype(jnp.int32)[:, None]
    w1v = jnp.max(probs, axis=1, keepdims=True)
    probs2 = jnp.where(ids == i1, -jnp.inf, probs)
    i2 = jnp.argmax(probs2, axis=1).astype(jnp.int32)[:, None]
    w2v = jnp.max(probs2, axis=1, keepdims=True)
    rw_ref[...] = jnp.concatenate([w1v, w2v], axis=1)
    se_ref[...] = jnp.concatenate([i1, i2], axis=1)


def _pos_body(se_ref, ps_ref, eof_ref, used_ref, run_ref, rs_ref):
    g = pl.program_id(0)
    seb = se_ref[...]
    ids = jax.lax.broadcasted_iota(jnp.int32, (TB, E), 1)
    oh0 = (ids == seb[:, 0:1]).astype(jnp.float32)
    oh1 = (ids == seb[:, 1:2]).astype(jnp.float32)
    ocnt = oh0 + oh1

    @pl.when(g == 0)
    def _():
        run_ref[...] = jnp.zeros((1, E), jnp.float32)

    @pl.when(g < NCH)
    def _():
        run_ref[...] += jnp.sum(ocnt, axis=0, keepdims=True)

    @pl.when(g == NCH)
    def _():
        totals = run_ref[...]
        tiles = jnp.floor((totals + (M - 1)) * (1.0 / M))
        ei = jax.lax.broadcasted_iota(jnp.int32, (E, E), 0)
        ej = jax.lax.broadcasted_iota(jnp.int32, (E, E), 1)
        lstrict_e = (ei < ej).astype(jnp.float32)
        ts_tiles = jax.lax.dot_general(
            tiles, lstrict_e, (((1,), (0,)), ((), ())),
            preferred_element_type=jnp.float32)
        rs_ref[...] = ts_tiles * float(M)
        run_ref[...] = jnp.zeros((1, E), jnp.float32)
        cum_incl = ts_tiles + tiles
        gi = jax.lax.broadcasted_iota(jnp.int32, (NPT, E), 0).astype(jnp.float32)
        eof = jnp.sum((gi >= cum_incl).astype(jnp.float32), axis=1,
                      keepdims=True)
        eof_ref[...] = jnp.minimum(eof, float(E - 1)).astype(jnp.int32)
        used_ref[...] = jnp.sum(tiles, axis=1, keepdims=True).astype(jnp.int32)

    @pl.when(g >= NCH)
    def _():
        ri = jax.lax.broadcasted_iota(jnp.int32, (TB, TB), 0)
        rj = jax.lax.broadcasted_iota(jnp.int32, (TB, TB), 1)
        lstrict = (rj < ri).astype(jnp.float32)
        crow = jax.lax.dot_general(
            lstrict, ocnt, (((1,), (0,)), ((), ())),
            preferred_element_type=jnp.float32)
        base = rs_ref[...] + run_ref[...] + crow
        p0 = jnp.sum(oh0 * base, axis=1, keepdims=True)
        p1 = jnp.sum(oh1 * (base + oh0), axis=1, keepdims=True)
        ps_ref[...] = jnp.concatenate([p0, p1], axis=1).astype(jnp.int32)
        run_ref[...] += jnp.sum(ocnt, axis=0, keepdims=True)


def _make_sc_dispatch():
    mesh = plsc.VectorSubcoreMesh(core_axis_name="c", subcore_axis_name="s")

    def body(x_hbm, ps0_hbm, ps1_hbm, xs_hbm, idx_v, rows_v, sem):
        wid = jax.lax.axis_index("s") * 2 + jax.lax.axis_index("c")
        for c in range(TPW // CCH):
            tbase = wid * TPW + c * CCH
            pltpu.sync_copy(x_hbm.at[pl.ds(tbase, CCH)], rows_v)
            pltpu.sync_copy(ps0_hbm.at[wid, c], idx_v)
            pltpu.async_copy(rows_v, xs_hbm.at[idx_v], sem).wait()
            pltpu.sync_copy(ps1_hbm.at[wid, c], idx_v)
            pltpu.async_copy(rows_v, xs_hbm.at[idx_v], sem).wait()

    return pl.kernel(
        body, mesh=mesh,
        out_type=jax.ShapeDtypeStruct((PAD, H), jnp.float32),
        scratch_types=[
            pltpu.VMEM((CCH,), jnp.int32),
            pltpu.VMEM((CCH, H), jnp.float32),
            pltpu.SemaphoreType.DMA,
        ],
    )


def _make_sc_combine():
    mesh = plsc.VectorSubcoreMesh(core_axis_name="c", subcore_axis_name="s")
    per_w = TK // NW
    n_ch = per_w // (2 * CCH)

    def body(ps_hbm, ys_hbm, ysg_hbm, idx_v, rows_v, sem):
        wid = jax.lax.axis_index("s") * 2 + jax.lax.axis_index("c")
        base = wid * per_w
        for c in range(n_ch):
            off = base + c * (2 * CCH)
            pltpu.sync_copy(ps_hbm.at[pl.ds(off, 2 * CCH)], idx_v)
            pltpu.async_copy(ys_hbm.at[idx_v], rows_v, sem).wait()
            pltpu.sync_copy(rows_v, ysg_hbm.at[pl.ds(off, 2 * CCH)])

    return pl.kernel(
        body, mesh=mesh,
        out_type=jax.ShapeDtypeStruct((TK, H), jnp.float32),
        scratch_types=[
            pltpu.VMEM((2 * CCH,), jnp.int32),
            pltpu.VMEM((2 * CCH, H), jnp.float32),
            pltpu.SemaphoreType.DMA,
        ],
    )


def _sc_dispatch(x, ps0, ps1):
    return _make_sc_dispatch()(x, ps0, ps1)


def _sc_combine(ps, ys):
    return _make_sc_combine()(ps, ys)


def _expert_body(e_of_ref, used_ref, xs_ref, w1_ref, w3_ref, w2_ref, ys_ref):
    @pl.when(pl.program_id(0) < used_ref[0])
    def _():
        xb = xs_ref[...].astype(jnp.bfloat16)
        w1b = w1_ref[0].astype(jnp.bfloat16)
        w3b = w3_ref[0].astype(jnp.bfloat16)
        w2b = w2_ref[0].astype(jnp.bfloat16)

        a = jax.lax.dot_general(xb, w1b, (((1,), (1,)), ((), ())),
                                preferred_element_type=jnp.float32)
        b = jax.lax.dot_general(xb, w3b, (((1,), (1,)), ((), ())),
                                preferred_element_type=jnp.float32)
        h = (a * jax.nn.sigmoid(a) * b).astype(jnp.bfloat16)
        ys_ref[...] = jax.lax.dot_general(h, w2b, (((1,), (1,)), ((), ())),
                                          preferred_element_type=jnp.float32)


def _combine_body(ysg_ref, rw_ref, out_ref):
    rwb = rw_ref[...]
    out_ref[...] = (rwb[:, 0:1] * ysg_ref[:, 0, :]
                    + rwb[:, 1:2] * ysg_ref[:, 1, :])


@jax.jit
def kernel(x, gate_w, w1, w2, w3):
    rw, se = pl.pallas_call(
        _router_body,
        grid=(NCH,),
        in_specs=[
            pl.BlockSpec((TB, H), lambda t: (t, 0)),
            pl.BlockSpec((E, H), lambda t: (0, 0)),
        ],
        out_specs=[
            pl.BlockSpec((TB, TOP_K), lambda t: (t, 0)),
            pl.BlockSpec((TB, TOP_K), lambda t: (t, 0)),
        ],
        out_shape=[
            jax.ShapeDtypeStruct((T, TOP_K), jnp.float32),
            jax.ShapeDtypeStruct((T, TOP_K), jnp.int32),
        ],
    )(x, gate_w)

    ps, e_of, used = pl.pallas_call(
        _pos_body,
        grid=(2 * NCH,),
        in_specs=[
            pl.BlockSpec((TB, TOP_K),
                         lambda g: (jnp.where(g < NCH, g, g - NCH), 0)),
        ],
        out_specs=[
            pl.BlockSpec((TB, TOP_K),
                         lambda g: (jnp.where(g < NCH, 0, g - NCH), 0)),
            pl.BlockSpec((NPT, 1), lambda g: (0, 0)),
            pl.BlockSpec((1, 1), lambda g: (0, 0)),
        ],
        out_shape=[
            jax.ShapeDtypeStruct((T, TOP_K), jnp.int32),
            jax.ShapeDtypeStruct((NPT, 1), jnp.int32),
            jax.ShapeDtypeStruct((1, 1), jnp.int32),
        ],
        scratch_shapes=[
            pltpu.VMEM((1, E), jnp.float32),
            pltpu.VMEM((1, E), jnp.float32),
        ],
    )(se)
    e_of = e_of.reshape(NPT)
    used = used.reshape(1)

    ps0 = ps[:, 0].reshape(NW, TPW // CCH, CCH)
    ps1 = ps[:, 1].reshape(NW, TPW // CCH, CCH)

    xs = _sc_dispatch(x, ps0, ps1)

    ys = pl.pallas_call(
        _expert_body,
        grid_spec=pltpu.PrefetchScalarGridSpec(
            num_scalar_prefetch=2,
            grid=(NPT,),
            in_specs=[
                pl.BlockSpec((M, H), lambda g, eo, u: (g, 0)),
                pl.BlockSpec((1, I, H), lambda g, eo, u: (eo[g], 0, 0)),
                pl.BlockSpec((1, I, H), lambda g, eo, u: (eo[g], 0, 0)),
                pl.BlockSpec((1, H, I), lambda g, eo, u: (eo[g], 0, 0)),
            ],
            out_specs=pl.BlockSpec((M, H), lambda g, eo, u: (g, 0)),
        ),
        out_shape=jax.ShapeDtypeStruct((PAD, H), jnp.float32),
    )(e_of, used, xs, w1, w3, w2)

    ysg = _sc_combine(ps.reshape(TK), ys).reshape(T, TOP_K, H)

    out = pl.pallas_call(
        _combine_body,
        grid=(NCH,),
        in_specs=[
            pl.BlockSpec((TB, TOP_K, H), lambda t: (t, 0, 0)),
            pl.BlockSpec((TB, TOP_K), lambda t: (t, 0)),
        ],
        out_specs=pl.BlockSpec((TB, H), lambda t: (t, 0)),
        out_shape=jax.ShapeDtypeStruct((T, H), jnp.float32),
    )(ysg, rw)
    return out


# speedup vs baseline: 1.7242x; 1.0109x over previous
import jax
import jax.numpy as jnp
from jax.experimental import pallas as pl
from jax.experimental.pallas import tpu as pltpu
from jax.experimental.pallas import tpu_sc as plsc

E = 64
H = 768
I = 1536
T = 2048
TOP_K = 2
TK = T * TOP_K

TB = 256
NCH = T // TB
M = 64
NPT = TK // M + E
PAD = NPT * M

NW = 32
TPW = T // NW
CCH = 32


def _route_pos_body(x_ref, gw_ref, rw_ref, ps_ref, eof_ref, used_ref,
                    run_ref, rs_ref, se_s, rw_s):
    g = pl.program_id(0)
    ids = jax.lax.broadcasted_iota(jnp.int32, (TB, E), 1)

    @pl.when(g == 0)
    def _():
        run_ref[...] = jnp.zeros((1, E), jnp.float32)

    @pl.when(g < NCH)
    def _():
        xb = x_ref[...]
        gw = gw_ref[...]
        logits = jax.lax.dot_general(xb, gw, (((1,), (1,)), ((), ())),
                                     preferred_element_type=jnp.float32)
        m = jnp.max(logits, axis=1, keepdims=True)
        ex = jnp.exp(logits - m)
        probs = ex / jnp.sum(ex, axis=1, keepdims=True)
        i1 = jnp.argmax(probs, axis=1).astype(jnp.int32)[:, None]
        w1v = jnp.max(probs, axis=1, keepdims=True)
        probs2 = jnp.where(ids == i1, -jnp.inf, probs)
        i2 = jnp.argmax(probs2, axis=1).astype(jnp.int32)[:, None]
        w2v = jnp.max(probs2, axis=1, keepdims=True)
        seb = jnp.concatenate([i1, i2], axis=1)
        se_s[g] = seb
        rw_s[g] = jnp.concatenate([w1v, w2v], axis=1)
        ocnt = ((ids == i1).astype(jnp.float32)
                + (ids == i2).astype(jnp.float32))
        run_ref[...] += jnp.sum(ocnt, axis=0, keepdims=True)

    @pl.when(g == NCH)
    def _():
        totals = run_ref[...]
        tiles = jnp.floor((totals + (M - 1)) * (1.0 / M))
        ei = jax.lax.broadcasted_iota(jnp.int32, (E, E), 0)
        ej = jax.lax.broadcasted_iota(jnp.int32, (E, E), 1)
        lstrict_e = (ei < ej).astype(jnp.float32)
        ts_tiles = jax.lax.dot_general(
            tiles, lstrict_e, (((1,), (0,)), ((), ())),
            preferred_element_type=jnp.float32)
        rs_ref[...] = ts_tiles * float(M)
        run_ref[...] = jnp.zeros((1, E), jnp.float32)
        cum_incl = ts_tiles + tiles
        gi = jax.lax.broadcasted_iota(jnp.int32, (NPT, E), 0).astype(jnp.float32)
        eof = jnp.sum((gi >= cum_incl).astype(jnp.float32), axis=1,
                      keepdims=True)
        eof_ref[...] = jnp.minimum(eof, float(E - 1)).astype(jnp.int32)
        used_ref[...] = jnp.sum(tiles, axis=1, keepdims=True).astype(jnp.int32)

    @pl.when(g >= NCH)
    def _():
        seb = se_s[g - NCH]
        oh0 = (ids == seb[:, 0:1]).astype(jnp.float32)
        oh1 = (ids == seb[:, 1:2]).astype(jnp.float32)
        ocnt = oh0 + oh1
        ri = jax.lax.broadcasted_iota(jnp.int32, (TB, TB), 0)
        rj = jax.lax.broadcasted_iota(jnp.int32, (TB, TB), 1)
        lstrict = (rj < ri).astype(jnp.float32)
        crow = jax.lax.dot_general(
            lstrict, ocnt, (((1,), (0,)), ((), ())),
            preferred_element_type=jnp.float32)
        base = rs_ref[...] + run_ref[...] + crow
        p0 = jnp.sum(oh0 * base, axis=1, keepdims=True)
        p1 = jnp.sum(oh1 * (base + oh0), axis=1, keepdims=True)
        ps_ref[...] = jnp.concatenate([p0, p1], axis=1).astype(jnp.int32)
        rw_ref[...] = rw_s[g - NCH]
        run_ref[...] += jnp.sum(ocnt, axis=0, keepdims=True)


def _make_sc_dispatch():
    mesh = plsc.VectorSubcoreMesh(core_axis_name="c", subcore_axis_name="s")

    def body(x_hbm, ps0_hbm, ps1_hbm, xs_hbm, idx_v, rows_v, sem):
        wid = jax.lax.axis_index("s") * 2 + jax.lax.axis_index("c")
        for c in range(TPW // CCH):
            tbase = wid * TPW + c * CCH
            pltpu.sync_copy(x_hbm.at[pl.ds(tbase, CCH)], rows_v)
            pltpu.sync_copy(ps0_hbm.at[wid, c], idx_v)
            pltpu.async_copy(rows_v, xs_hbm.at[idx_v], sem).wait()
            pltpu.sync_copy(ps1_hbm.at[wid, c], idx_v)
            pltpu.async_copy(rows_v, xs_hbm.at[idx_v], sem).wait()

    return pl.kernel(
        body, mesh=mesh,
        out_type=jax.ShapeDtypeStruct((PAD, H), jnp.float32),
        scratch_types=[
            pltpu.VMEM((CCH,), jnp.int32),
            pltpu.VMEM((CCH, H), jnp.float32),
            pltpu.SemaphoreType.DMA,
        ],
    )


def _make_sc_combine():
    mesh = plsc.VectorSubcoreMesh(core_axis_name="c", subcore_axis_name="s")
    per_w = TK // NW
    n_ch = per_w // (2 * CCH)

    def body(ps_hbm, ys_hbm, ysg_hbm, idx_v, rows_v, sem):
        wid = jax.lax.axis_index("s") * 2 + jax.lax.axis_index("c")
        base = wid * per_w
        for c in range(n_ch):
            off = base + c * (2 * CCH)
            pltpu.sync_copy(ps_hbm.at[pl.ds(off, 2 * CCH)], idx_v)
            pltpu.async_copy(ys_hbm.at[idx_v], rows_v, sem).wait()
            pltpu.sync_copy(rows_v, ysg_hbm.at[pl.ds(off, 2 * CCH)])

    return pl.kernel(
        body, mesh=mesh,
        out_type=jax.ShapeDtypeStruct((TK, H), jnp.float32),
        scratch_types=[
            pltpu.VMEM((2 * CCH,), jnp.int32),
            pltpu.VMEM((2 * CCH, H), jnp.float32),
            pltpu.SemaphoreType.DMA,
        ],
    )


def _sc_dispatch(x, ps0, ps1):
    return _make_sc_dispatch()(x, ps0, ps1)


def _sc_combine(ps, ys):
    return _make_sc_combine()(ps, ys)


def _expert_body(e_of_ref, used_ref, xs_ref, w1_ref, w3_ref, w2_ref, ys_ref):
    @pl.when(pl.program_id(0) < used_ref[0])
    def _():
        xb = xs_ref[...].astype(jnp.bfloat16)
        w1b = w1_ref[0].astype(jnp.bfloat16)
        w3b = w3_ref[0].astype(jnp.bfloat16)
        w2b = w2_ref[0].astype(jnp.bfloat16)

        a = jax.lax.dot_general(xb, w1b, (((1,), (1,)), ((), ())),
                                preferred_element_type=jnp.float32)
        b = jax.lax.dot_general(xb, w3b, (((1,), (1,)), ((), ())),
                                preferred_element_type=jnp.float32)
        h = (a * jax.nn.sigmoid(a) * b).astype(jnp.bfloat16)
        ys_ref[...] = jax.lax.dot_general(h, w2b, (((1,), (1,)), ((), ())),
                                          preferred_element_type=jnp.float32)


def _combine_body(ysg_ref, rw_ref, out_ref):
    rwb = rw_ref[...]
    out_ref[...] = (rwb[:, 0:1] * ysg_ref[:, 0, :]
                    + rwb[:, 1:2] * ysg_ref[:, 1, :])


@jax.jit
def kernel(x, gate_w, w1, w2, w3):
    rw, ps, e_of, used = pl.pallas_call(
        _route_pos_body,
        grid=(2 * NCH,),
        in_specs=[
            pl.BlockSpec((TB, H),
                         lambda g: (jnp.where(g < NCH, g, g - NCH), 0)),
            pl.BlockSpec((E, H), lambda g: (0, 0)),
        ],
        out_specs=[
            pl.BlockSpec((TB, TOP_K),
                         lambda g: (jnp.where(g < NCH, 0, g - NCH), 0)),
            pl.BlockSpec((TB, TOP_K),
                         lambda g: (jnp.where(g < NCH, 0, g - NCH), 0)),
            pl.BlockSpec((NPT, 1), lambda g: (0, 0)),
            pl.BlockSpec((1, 1), lambda g: (0, 0)),
        ],
        out_shape=[
            jax.ShapeDtypeStruct((T, TOP_K), jnp.float32),
            jax.ShapeDtypeStruct((T, TOP_K), jnp.int32),
            jax.ShapeDtypeStruct((NPT, 1), jnp.int32),
            jax.ShapeDtypeStruct((1, 1), jnp.int32),
        ],
        scratch_shapes=[
            pltpu.VMEM((1, E), jnp.float32),
            pltpu.VMEM((1, E), jnp.float32),
            pltpu.VMEM((NCH, TB, TOP_K), jnp.int32),
            pltpu.VMEM((NCH, TB, TOP_K), jnp.float32),
        ],
    )(x, gate_w)
    e_of = e_of.reshape(NPT)
    used = used.reshape(1)

    ps0 = ps[:, 0].reshape(NW, TPW // CCH, CCH)
    ps1 = ps[:, 1].reshape(NW, TPW // CCH, CCH)

    xs = _sc_dispatch(x, ps0, ps1)

    ys = pl.pallas_call(
        _expert_body,
        grid_spec=pltpu.PrefetchScalarGridSpec(
            num_scalar_prefetch=2,
            grid=(NPT,),
            in_specs=[
                pl.BlockSpec((M, H), lambda g, eo, u: (g, 0)),
                pl.BlockSpec((1, I, H), lambda g, eo, u: (eo[g], 0, 0)),
                pl.BlockSpec((1, I, H), lambda g, eo, u: (eo[g], 0, 0)),
                pl.BlockSpec((1, H, I), lambda g, eo, u: (eo[g], 0, 0)),
            ],
            out_specs=pl.BlockSpec((M, H), lambda g, eo, u: (g, 0)),
        ),
        out_shape=jax.ShapeDtypeStruct((PAD, H), jnp.float32),
    )(e_of, used, xs, w1, w3, w2)

    ysg = _sc_combine(ps.reshape(TK), ys).reshape(T, TOP_K, H)

    out = pl.pallas_call(
        _combine_body,
        grid=(NCH,),
        in_specs=[
            pl.BlockSpec((TB, TOP_K, H), lambda t: (t, 0, 0)),
            pl.BlockSpec((TB, TOP_K), lambda t: (t, 0)),
        ],
        out_specs=pl.BlockSpec((TB, H), lambda t: (t, 0)),
        out_shape=jax.ShapeDtypeStruct((T, H), jnp.float32),
    )(ysg, rw)
    return out


# speedup vs baseline: 1.7378x; 1.0079x over previous
import jax
import jax.numpy as jnp
from jax.experimental import pallas as pl
from jax.experimental.pallas import tpu as pltpu
from jax.experimental.pallas import tpu_sc as plsc

E = 64
H = 768
I = 1536
T = 2048
TOP_K = 2
TK = T * TOP_K

TB = 512
NCH = T // TB
M = 64
NPT = TK // M + E
PAD = NPT * M

NW = 32
TPW = T // NW
CCH = 32


def _route_pos_body(x_ref, gw_ref, rw_ref, ps_ref, eof_ref, used_ref,
                    run_ref, rs_ref, se_s, rw_s):
    g = pl.program_id(0)
    ids = jax.lax.broadcasted_iota(jnp.int32, (TB, E), 1)

    @pl.when(g == 0)
    def _():
        run_ref[...] = jnp.zeros((1, E), jnp.float32)

    @pl.when(g < NCH)
    def _():
        xb = x_ref[...]
        gw = gw_ref[...]
        logits = jax.lax.dot_general(xb, gw, (((1,), (1,)), ((), ())),
                                     preferred_element_type=jnp.float32)
        m = jnp.max(logits, axis=1, keepdims=True)
        ex = jnp.exp(logits - m)
        probs = ex / jnp.sum(ex, axis=1, keepdims=True)
        i1 = jnp.argmax(probs, axis=1).astype(jnp.int32)[:, None]
        w1v = jnp.max(probs, axis=1, keepdims=True)
        probs2 = jnp.where(ids == i1, -jnp.inf, probs)
        i2 = jnp.argmax(probs2, axis=1).astype(jnp.int32)[:, None]
        w2v = jnp.max(probs2, axis=1, keepdims=True)
        seb = jnp.concatenate([i1, i2], axis=1)
        se_s[g] = seb
        rw_s[g] = jnp.concatenate([w1v, w2v], axis=1)
        ocnt = ((ids == i1).astype(jnp.float32)
                + (ids == i2).astype(jnp.float32))
        run_ref[...] += jnp.sum(ocnt, axis=0, keepdims=True)

    @pl.when(g == NCH)
    def _():
        totals = run_ref[...]
        tiles = jnp.floor((totals + (M - 1)) * (1.0 / M))
        ei = jax.lax.broadcasted_iota(jnp.int32, (E, E), 0)
        ej = jax.lax.broadcasted_iota(jnp.int32, (E, E), 1)
        lstrict_e = (ei < ej).astype(jnp.float32)
        ts_tiles = jax.lax.dot_general(
            tiles, lstrict_e, (((1,), (0,)), ((), ())),
            preferred_element_type=jnp.float32)
        rs_ref[...] = ts_tiles * float(M)
        run_ref[...] = jnp.zeros((1, E), jnp.float32)
        cum_incl = ts_tiles + tiles
        gi = jax.lax.broadcasted_iota(jnp.int32, (NPT, E), 0).astype(jnp.float32)
        eof = jnp.sum((gi >= cum_incl).astype(jnp.float32), axis=1,
                      keepdims=True)
        eof_ref[...] = jnp.minimum(eof, float(E - 1)).astype(jnp.int32)
        used_ref[...] = jnp.sum(tiles, axis=1, keepdims=True).astype(jnp.int32)

    @pl.when(g >= NCH)
    def _():
        seb = se_s[g - NCH]
        oh0 = (ids == seb[:, 0:1]).astype(jnp.float32)
        oh1 = (ids == seb[:, 1:2]).astype(jnp.float32)
        ocnt = oh0 + oh1
        ri = jax.lax.broadcasted_iota(jnp.int32, (TB, TB), 0)
        rj = jax.lax.broadcasted_iota(jnp.int32, (TB, TB), 1)
        lstrict = (rj < ri).astype(jnp.float32)
        crow = jax.lax.dot_general(
            lstrict, ocnt, (((1,), (0,)), ((), ())),
            preferred_element_type=jnp.float32)
        base = rs_ref[...] + run_ref[...] + crow
        p0 = jnp.sum(oh0 * base, axis=1, keepdims=True)
        p1 = jnp.sum(oh1 * (base + oh0), axis=1, keepdims=True)
        ps_ref[...] = jnp.concatenate([p0, p1], axis=1).astype(jnp.int32)
        rw_ref[...] = rw_s[g - NCH]
        run_ref[...] += jnp.sum(ocnt, axis=0, keepdims=True)


def _make_sc_dispatch():
    mesh = plsc.VectorSubcoreMesh(core_axis_name="c", subcore_axis_name="s")

    def body(x_hbm, ps0_hbm, ps1_hbm, xs_hbm, idx_v, rows_v, sem):
        wid = jax.lax.axis_index("s") * 2 + jax.lax.axis_index("c")
        for c in range(TPW // CCH):
            tbase = wid * TPW + c * CCH
            pltpu.sync_copy(x_hbm.at[pl.ds(tbase, CCH)], rows_v)
            pltpu.sync_copy(ps0_hbm.at[wid, c], idx_v)
            pltpu.async_copy(rows_v, xs_hbm.at[idx_v], sem).wait()
            pltpu.sync_copy(ps1_hbm.at[wid, c], idx_v)
            pltpu.async_copy(rows_v, xs_hbm.at[idx_v], sem).wait()

    return pl.kernel(
        body, mesh=mesh,
        out_type=jax.ShapeDtypeStruct((PAD, H), jnp.float32),
        scratch_types=[
            pltpu.VMEM((CCH,), jnp.int32),
            pltpu.VMEM((CCH, H), jnp.float32),
            pltpu.SemaphoreType.DMA,
        ],
    )


def _make_sc_combine():
    mesh = plsc.VectorSubcoreMesh(core_axis_name="c", subcore_axis_name="s")
    per_w = TK // NW
    n_ch = per_w // (2 * CCH)

    def body(ps_hbm, ys_hbm, ysg_hbm, idx_v, rows_v, sem):
        wid = jax.lax.axis_index("s") * 2 + jax.lax.axis_index("c")
        base = wid * per_w
        for c in range(n_ch):
            off = base + c * (2 * CCH)
            pltpu.sync_copy(ps_hbm.at[pl.ds(off, 2 * CCH)], idx_v)
            pltpu.async_copy(ys_hbm.at[idx_v], rows_v, sem).wait()
            pltpu.sync_copy(rows_v, ysg_hbm.at[pl.ds(off, 2 * CCH)])

    return pl.kernel(
        body, mesh=mesh,
        out_type=jax.ShapeDtypeStruct((TK, H), jnp.float32),
        scratch_types=[
            pltpu.VMEM((2 * CCH,), jnp.int32),
            pltpu.VMEM((2 * CCH, H), jnp.float32),
            pltpu.SemaphoreType.DMA,
        ],
    )


def _sc_dispatch(x, ps0, ps1):
    return _make_sc_dispatch()(x, ps0, ps1)


def _sc_combine(ps, ys):
    return _make_sc_combine()(ps, ys)


def _expert_body(e_of_ref, used_ref, xs_ref, w1_ref, w3_ref, w2_ref, ys_ref):
    @pl.when(pl.program_id(0) < used_ref[0])
    def _():
        xb = xs_ref[...].astype(jnp.bfloat16)
        w1b = w1_ref[0].astype(jnp.bfloat16)
        w3b = w3_ref[0].astype(jnp.bfloat16)
        w2b = w2_ref[0].astype(jnp.bfloat16)

        a = jax.lax.dot_general(xb, w1b, (((1,), (1,)), ((), ())),
                                preferred_element_type=jnp.float32)
        b = jax.lax.dot_general(xb, w3b, (((1,), (1,)), ((), ())),
                                preferred_element_type=jnp.float32)
        h = (a * jax.nn.sigmoid(a) * b).astype(jnp.bfloat16)
        ys_ref[...] = jax.lax.dot_general(h, w2b, (((1,), (1,)), ((), ())),
                                          preferred_element_type=jnp.float32)


def _combine_body(ysg_ref, rw_ref, out_ref):
    rwb = rw_ref[...]
    out_ref[...] = (rwb[:, 0:1] * ysg_ref[:, 0, :]
                    + rwb[:, 1:2] * ysg_ref[:, 1, :])


@jax.jit
def kernel(x, gate_w, w1, w2, w3):
    rw, ps, e_of, used = pl.pallas_call(
        _route_pos_body,
        grid=(2 * NCH,),
        in_specs=[
            pl.BlockSpec((TB, H),
                         lambda g: (jnp.where(g < NCH, g, g - NCH), 0)),
            pl.BlockSpec((E, H), lambda g: (0, 0)),
        ],
        out_specs=[
            pl.BlockSpec((TB, TOP_K),
                         lambda g: (jnp.where(g < NCH, 0, g - NCH), 0)),
            pl.BlockSpec((TB, TOP_K),
                         lambda g: (jnp.where(g < NCH, 0, g - NCH), 0)),
            pl.BlockSpec((NPT, 1), lambda g: (0, 0)),
            pl.BlockSpec((1, 1), lambda g: (0, 0)),
        ],
        out_shape=[
            jax.ShapeDtypeStruct((T, TOP_K), jnp.float32),
            jax.ShapeDtypeStruct((T, TOP_K), jnp.int32),
            jax.ShapeDtypeStruct((NPT, 1), jnp.int32),
            jax.ShapeDtypeStruct((1, 1), jnp.int32),
        ],
        scratch_shapes=[
            pltpu.VMEM((1, E), jnp.float32),
            pltpu.VMEM((1, E), jnp.float32),
            pltpu.VMEM((NCH, TB, TOP_K), jnp.int32),
            pltpu.VMEM((NCH, TB, TOP_K), jnp.float32),
        ],
    )(x, gate_w)
    e_of = e_of.reshape(NPT)
    used = used.reshape(1)

    ps0 = ps[:, 0].reshape(NW, TPW // CCH, CCH)
    ps1 = ps[:, 1].reshape(NW, TPW // CCH, CCH)

    xs = _sc_dispatch(x, ps0, ps1)

    ys = pl.pallas_call(
        _expert_body,
        grid_spec=pltpu.PrefetchScalarGridSpec(
            num_scalar_prefetch=2,
            grid=(NPT,),
            in_specs=[
                pl.BlockSpec((M, H), lambda g, eo, u: (g, 0)),
                pl.BlockSpec((1, I, H), lambda g, eo, u: (eo[g], 0, 0)),
                pl.BlockSpec((1, I, H), lambda g, eo, u: (eo[g], 0, 0)),
                pl.BlockSpec((1, H, I), lambda g, eo, u: (eo[g], 0, 0)),
            ],
            out_specs=pl.BlockSpec((M, H), lambda g, eo, u: (g, 0)),
        ),
        out_shape=jax.ShapeDtypeStruct((PAD, H), jnp.float32),
    )(e_of, used, xs, w1, w3, w2)

    ysg = _sc_combine(ps.reshape(TK), ys).reshape(T, TOP_K, H)

    out = pl.pallas_call(
        _combine_body,
        grid=(NCH,),
        in_specs=[
            pl.BlockSpec((TB, TOP_K, H), lambda t: (t, 0, 0)),
            pl.BlockSpec((TB, TOP_K), lambda t: (t, 0)),
        ],
        out_specs=pl.BlockSpec((TB, H), lambda t: (t, 0)),
        out_shape=jax.ShapeDtypeStruct((T, H), jnp.float32),
    )(ysg, rw)
    return out


# speedup vs baseline: 2.1640x; 1.2453x over previous
import jax
import jax.numpy as jnp
from jax.experimental import pallas as pl
from jax.experimental.pallas import tpu as pltpu
from jax.experimental.pallas import tpu_sc as plsc

E = 64
H = 768
I = 1536
T = 2048
TOP_K = 2
TK = T * TOP_K

TB = 512
NCH = T // TB
M = 128
NPT = TK // M + E
PAD = NPT * M

NW = 32
TPW = T // NW
CCH = 32


def _route_pos_body(x_ref, gw_ref, rw_ref, ps_ref, eof_ref, used_ref,
                    run_ref, rs_ref, se_s, rw_s):
    g = pl.program_id(0)
    ids = jax.lax.broadcasted_iota(jnp.int32, (TB, E), 1)

    @pl.when(g == 0)
    def _():
        run_ref[...] = jnp.zeros((1, E), jnp.float32)

    @pl.when(g < NCH)
    def _():
        xb = x_ref[...]
        gw = gw_ref[...]
        logits = jax.lax.dot_general(xb, gw, (((1,), (1,)), ((), ())),
                                     preferred_element_type=jnp.float32)
        m = jnp.max(logits, axis=1, keepdims=True)
        ex = jnp.exp(logits - m)
        probs = ex / jnp.sum(ex, axis=1, keepdims=True)
        i1 = jnp.argmax(probs, axis=1).astype(jnp.int32)[:, None]
        w1v = jnp.max(probs, axis=1, keepdims=True)
        probs2 = jnp.where(ids == i1, -jnp.inf, probs)
        i2 = jnp.argmax(probs2, axis=1).astype(jnp.int32)[:, None]
        w2v = jnp.max(probs2, axis=1, keepdims=True)
        seb = jnp.concatenate([i1, i2], axis=1)
        se_s[g] = seb
        rw_s[g] = jnp.concatenate([w1v, w2v], axis=1)
        ocnt = ((ids == i1).astype(jnp.float32)
                + (ids == i2).astype(jnp.float32))
        run_ref[...] += jnp.sum(ocnt, axis=0, keepdims=True)

    @pl.when(g == NCH)
    def _():
        totals = run_ref[...]
        tiles = jnp.floor((totals + (M - 1)) * (1.0 / M))
        ei = jax.lax.broadcasted_iota(jnp.int32, (E, E), 0)
        ej = jax.lax.broadcasted_iota(jnp.int32, (E, E), 1)
        lstrict_e = (ei < ej).astype(jnp.float32)
        ts_tiles = jax.lax.dot_general(
            tiles, lstrict_e, (((1,), (0,)), ((), ())),
            preferred_element_type=jnp.float32)
        rs_ref[...] = ts_tiles * float(M)
        run_ref[...] = jnp.zeros((1, E), jnp.float32)
        cum_incl = ts_tiles + tiles
        gi = jax.lax.broadcasted_iota(jnp.int32, (NPT, E), 0).astype(jnp.float32)
        eof = jnp.sum((gi >= cum_incl).astype(jnp.float32), axis=1,
                      keepdims=True)
        eof_ref[...] = jnp.minimum(eof, float(E - 1)).astype(jnp.int32)
        used_ref[...] = jnp.sum(tiles, axis=1, keepdims=True).astype(jnp.int32)

    @pl.when(g >= NCH)
    def _():
        seb = se_s[g - NCH]
        oh0 = (ids == seb[:, 0:1]).astype(jnp.float32)
        oh1 = (ids == seb[:, 1:2]).astype(jnp.float32)
        ocnt = oh0 + oh1
        ri = jax.lax.broadcasted_iota(jnp.int32, (TB, TB), 0)
        rj = jax.lax.broadcasted_iota(jnp.int32, (TB, TB), 1)
        lstrict = (rj < ri).astype(jnp.float32)
        crow = jax.lax.dot_general(
            lstrict, ocnt, (((1,), (0,)), ((), ())),
            preferred_element_type=jnp.float32)
        base = rs_ref[...] + run_ref[...] + crow
        p0 = jnp.sum(oh0 * base, axis=1, keepdims=True)
        p1 = jnp.sum(oh1 * (base + oh0), axis=1, keepdims=True)
        ps_ref[...] = jnp.concatenate([p0, p1], axis=1).astype(jnp.int32)
        rw_ref[...] = rw_s[g - NCH]
        run_ref[...] += jnp.sum(ocnt, axis=0, keepdims=True)


def _make_sc_dispatch():
    mesh = plsc.VectorSubcoreMesh(core_axis_name="c", subcore_axis_name="s")

    def body(x_hbm, ps0_hbm, ps1_hbm, xs_hbm, idx_v, rows_v, sem):
        wid = jax.lax.axis_index("s") * 2 + jax.lax.axis_index("c")
        for c in range(TPW // CCH):
            tbase = wid * TPW + c * CCH
            pltpu.sync_copy(x_hbm.at[pl.ds(tbase, CCH)], rows_v)
            pltpu.sync_copy(ps0_hbm.at[wid, c], idx_v)
            pltpu.async_copy(rows_v, xs_hbm.at[idx_v], sem).wait()
            pltpu.sync_copy(ps1_hbm.at[wid, c], idx_v)
            pltpu.async_copy(rows_v, xs_hbm.at[idx_v], sem).wait()

    return pl.kernel(
        body, mesh=mesh,
        out_type=jax.ShapeDtypeStruct((PAD, H), jnp.float32),
        scratch_types=[
            pltpu.VMEM((CCH,), jnp.int32),
            pltpu.VMEM((CCH, H), jnp.float32),
            pltpu.SemaphoreType.DMA,
        ],
    )


def _make_sc_combine():
    mesh = plsc.VectorSubcoreMesh(core_axis_name="c", subcore_axis_name="s")
    per_w = TK // NW
    n_ch = per_w // (2 * CCH)

    def body(ps_hbm, ys_hbm, ysg_hbm, idx_v, rows_v, sem):
        wid = jax.lax.axis_index("s") * 2 + jax.lax.axis_index("c")
        base = wid * per_w
        for c in range(n_ch):
            off = base + c * (2 * CCH)
            pltpu.sync_copy(ps_hbm.at[pl.ds(off, 2 * CCH)], idx_v)
            pltpu.async_copy(ys_hbm.at[idx_v], rows_v, sem).wait()
            pltpu.sync_copy(rows_v, ysg_hbm.at[pl.ds(off, 2 * CCH)])

    return pl.kernel(
        body, mesh=mesh,
        out_type=jax.ShapeDtypeStruct((TK, H), jnp.float32),
        scratch_types=[
            pltpu.VMEM((2 * CCH,), jnp.int32),
            pltpu.VMEM((2 * CCH, H), jnp.float32),
            pltpu.SemaphoreType.DMA,
        ],
    )


def _sc_dispatch(x, ps0, ps1):
    return _make_sc_dispatch()(x, ps0, ps1)


def _sc_combine(ps, ys):
    return _make_sc_combine()(ps, ys)


def _expert_body(e_of_ref, used_ref, xs_ref, w1_ref, w3_ref, w2_ref, ys_ref):
    @pl.when(pl.program_id(0) < used_ref[0])
    def _():
        xb = xs_ref[...].astype(jnp.bfloat16)
        w1b = w1_ref[0].astype(jnp.bfloat16)
        w3b = w3_ref[0].astype(jnp.bfloat16)
        w2b = w2_ref[0].astype(jnp.bfloat16)

        a = jax.lax.dot_general(xb, w1b, (((1,), (1,)), ((), ())),
                                preferred_element_type=jnp.float32)
        b = jax.lax.dot_general(xb, w3b, (((1,), (1,)), ((), ())),
                                preferred_element_type=jnp.float32)
        h = (a * jax.nn.sigmoid(a) * b).astype(jnp.bfloat16)
        ys_ref[...] = jax.lax.dot_general(h, w2b, (((1,), (1,)), ((), ())),
                                          preferred_element_type=jnp.float32)


def _combine_body(ysg_ref, rw_ref, out_ref):
    rwb = rw_ref[...]
    out_ref[...] = (rwb[:, 0:1] * ysg_ref[:, 0, :]
                    + rwb[:, 1:2] * ysg_ref[:, 1, :])


@jax.jit
def kernel(x, gate_w, w1, w2, w3):
    rw, ps, e_of, used = pl.pallas_call(
        _route_pos_body,
        grid=(2 * NCH,),
        in_specs=[
            pl.BlockSpec((TB, H),
                         lambda g: (jnp.where(g < NCH, g, g - NCH), 0)),
            pl.BlockSpec((E, H), lambda g: (0, 0)),
        ],
        out_specs=[
            pl.BlockSpec((TB, TOP_K),
                         lambda g: (jnp.where(g < NCH, 0, g - NCH), 0)),
            pl.BlockSpec((TB, TOP_K),
                         lambda g: (jnp.where(g < NCH, 0, g - NCH), 0)),
            pl.BlockSpec((NPT, 1), lambda g: (0, 0)),
            pl.BlockSpec((1, 1), lambda g: (0, 0)),
        ],
        out_shape=[
            jax.ShapeDtypeStruct((T, TOP_K), jnp.float32),
            jax.ShapeDtypeStruct((T, TOP_K), jnp.int32),
            jax.ShapeDtypeStruct((NPT, 1), jnp.int32),
            jax.ShapeDtypeStruct((1, 1), jnp.int32),
        ],
        scratch_shapes=[
            pltpu.VMEM((1, E), jnp.float32),
            pltpu.VMEM((1, E), jnp.float32),
            pltpu.VMEM((NCH, TB, TOP_K), jnp.int32),
            pltpu.VMEM((NCH, TB, TOP_K), jnp.float32),
        ],
    )(x, gate_w)
    e_of = e_of.reshape(NPT)
    used = used.reshape(1)

    ps0 = ps[:, 0].reshape(NW, TPW // CCH, CCH)
    ps1 = ps[:, 1].reshape(NW, TPW // CCH, CCH)

    xs = _sc_dispatch(x, ps0, ps1)

    ys = pl.pallas_call(
        _expert_body,
        grid_spec=pltpu.PrefetchScalarGridSpec(
            num_scalar_prefetch=2,
            grid=(NPT,),
            in_specs=[
                pl.BlockSpec((M, H), lambda g, eo, u: (g, 0)),
                pl.BlockSpec((1, I, H), lambda g, eo, u: (eo[g], 0, 0)),
                pl.BlockSpec((1, I, H), lambda g, eo, u: (eo[g], 0, 0)),
                pl.BlockSpec((1, H, I), lambda g, eo, u: (eo[g], 0, 0)),
            ],
            out_specs=pl.BlockSpec((M, H), lambda g, eo, u: (g, 0)),
        ),
        out_shape=jax.ShapeDtypeStruct((PAD, H), jnp.float32),
    )(e_of, used, xs, w1, w3, w2)

    ysg = _sc_combine(ps.reshape(TK), ys).reshape(T, TOP_K, H)

    out = pl.pallas_call(
        _combine_body,
        grid=(NCH,),
        in_specs=[
            pl.BlockSpec((TB, TOP_K, H), lambda t: (t, 0, 0)),
            pl.BlockSpec((TB, TOP_K), lambda t: (t, 0)),
        ],
        out_specs=pl.BlockSpec((TB, H), lambda t: (t, 0)),
        out_shape=jax.ShapeDtypeStruct((T, H), jnp.float32),
    )(ysg, rw)
    return out


# speedup vs baseline: 2.1640x; 1.0000x over previous
import jax
import jax.numpy as jnp
from jax.experimental import pallas as pl
from jax.experimental.pallas import tpu as pltpu
from jax.experimental.pallas import tpu_sc as plsc

E = 64
H = 768
I = 1536
T = 2048
TOP_K = 2
TK = T * TOP_K

TB = 512
NCH = T // TB
M = 144
NPT = TK // M + E
PAD = NPT * M

NW = 32
TPW = T // NW
CCH = 32


def _route_pos_body(x_ref, gw_ref, rw_ref, ps_ref, eof_ref, used_ref,
                    run_ref, rs_ref, se_s, rw_s):
    g = pl.program_id(0)
    ids = jax.lax.broadcasted_iota(jnp.int32, (TB, E), 1)

    @pl.when(g == 0)
    def _():
        run_ref[...] = jnp.zeros((1, E), jnp.float32)

    @pl.when(g < NCH)
    def _():
        xb = x_ref[...]
        gw = gw_ref[...]
        logits = jax.lax.dot_general(xb, gw, (((1,), (1,)), ((), ())),
                                     preferred_element_type=jnp.float32)
        m = jnp.max(logits, axis=1, keepdims=True)
        ex = jnp.exp(logits - m)
        probs = ex / jnp.sum(ex, axis=1, keepdims=True)
        i1 = jnp.argmax(probs, axis=1).astype(jnp.int32)[:, None]
        w1v = jnp.max(probs, axis=1, keepdims=True)
        probs2 = jnp.where(ids == i1, -jnp.inf, probs)
        i2 = jnp.argmax(probs2, axis=1).astype(jnp.int32)[:, None]
        w2v = jnp.max(probs2, axis=1, keepdims=True)
        seb = jnp.concatenate([i1, i2], axis=1)
        se_s[g] = seb
        rw_s[g] = jnp.concatenate([w1v, w2v], axis=1)
        ocnt = ((ids == i1).astype(jnp.float32)
                + (ids == i2).astype(jnp.float32))
        run_ref[...] += jnp.sum(ocnt, axis=0, keepdims=True)

    @pl.when(g == NCH)
    def _():
        totals = run_ref[...]
        tiles = jnp.floor((totals + (M - 1)) * (1.0 / M))
        ei = jax.lax.broadcasted_iota(jnp.int32, (E, E), 0)
        ej = jax.lax.broadcasted_iota(jnp.int32, (E, E), 1)
        lstrict_e = (ei < ej).astype(jnp.float32)
        ts_tiles = jax.lax.dot_general(
            tiles, lstrict_e, (((1,), (0,)), ((), ())),
            preferred_element_type=jnp.float32)
        rs_ref[...] = ts_tiles * float(M)
        run_ref[...] = jnp.zeros((1, E), jnp.float32)
        cum_incl = ts_tiles + tiles
        gi = jax.lax.broadcasted_iota(jnp.int32, (NPT, E), 0).astype(jnp.float32)
        eof = jnp.sum((gi >= cum_incl).astype(jnp.float32), axis=1,
                      keepdims=True)
        eof_ref[...] = jnp.minimum(eof, float(E - 1)).astype(jnp.int32)
        used_ref[...] = jnp.sum(tiles, axis=1, keepdims=True).astype(jnp.int32)

    @pl.when(g >= NCH)
    def _():
        seb = se_s[g - NCH]
        oh0 = (ids == seb[:, 0:1]).astype(jnp.float32)
        oh1 = (ids == seb[:, 1:2]).astype(jnp.float32)
        ocnt = oh0 + oh1
        ri = jax.lax.broadcasted_iota(jnp.int32, (TB, TB), 0)
        rj = jax.lax.broadcasted_iota(jnp.int32, (TB, TB), 1)
        lstrict = (rj < ri).astype(jnp.float32)
        crow = jax.lax.dot_general(
            lstrict, ocnt, (((1,), (0,)), ((), ())),
            preferred_element_type=jnp.float32)
        base = rs_ref[...] + run_ref[...] + crow
        p0 = jnp.sum(oh0 * base, axis=1, keepdims=True)
        p1 = jnp.sum(oh1 * (base + oh0), axis=1, keepdims=True)
        ps_ref[...] = jnp.concatenate([p0, p1], axis=1).astype(jnp.int32)
        rw_ref[...] = rw_s[g - NCH]
        run_ref[...] += jnp.sum(ocnt, axis=0, keepdims=True)


def _make_sc_dispatch():
    mesh = plsc.VectorSubcoreMesh(core_axis_name="c", subcore_axis_name="s")

    def body(x_hbm, ps0_hbm, ps1_hbm, xs_hbm, idx_v, rows_v, sem):
        wid = jax.lax.axis_index("s") * 2 + jax.lax.axis_index("c")
        for c in range(TPW // CCH):
            tbase = wid * TPW + c * CCH
            pltpu.sync_copy(x_hbm.at[pl.ds(tbase, CCH)], rows_v)
            pltpu.sync_copy(ps0_hbm.at[wid, c], idx_v)
            pltpu.async_copy(rows_v, xs_hbm.at[idx_v], sem).wait()
            pltpu.sync_copy(ps1_hbm.at[wid, c], idx_v)
            pltpu.async_copy(rows_v, xs_hbm.at[idx_v], sem).wait()

    return pl.kernel(
        body, mesh=mesh,
        out_type=jax.ShapeDtypeStruct((PAD, H), jnp.float32),
        scratch_types=[
            pltpu.VMEM((CCH,), jnp.int32),
            pltpu.VMEM((CCH, H), jnp.float32),
            pltpu.SemaphoreType.DMA,
        ],
    )


def _make_sc_combine():
    mesh = plsc.VectorSubcoreMesh(core_axis_name="c", subcore_axis_name="s")
    per_w = TK // NW
    n_ch = per_w // (2 * CCH)

    def body(ps_hbm, ys_hbm, ysg_hbm, idx_v, rows_v, sem):
        wid = jax.lax.axis_index("s") * 2 + jax.lax.axis_index("c")
        base = wid * per_w
        for c in range(n_ch):
            off = base + c * (2 * CCH)
            pltpu.sync_copy(ps_hbm.at[pl.ds(off, 2 * CCH)], idx_v)
            pltpu.async_copy(ys_hbm.at[idx_v], rows_v, sem).wait()
            pltpu.sync_copy(rows_v, ysg_hbm.at[pl.ds(off, 2 * CCH)])

    return pl.kernel(
        body, mesh=mesh,
        out_type=jax.ShapeDtypeStruct((TK, H), jnp.float32),
        scratch_types=[
            pltpu.VMEM((2 * CCH,), jnp.int32),
            pltpu.VMEM((2 * CCH, H), jnp.float32),
            pltpu.SemaphoreType.DMA,
        ],
    )


def _sc_dispatch(x, ps0, ps1):
    return _make_sc_dispatch()(x, ps0, ps1)


def _sc_combine(ps, ys):
    return _make_sc_combine()(ps, ys)


def _expert_body(e_of_ref, used_ref, xs_ref, w1_ref, w3_ref, w2_ref, ys_ref):
    @pl.when(pl.program_id(0) < used_ref[0])
    def _():
        xb = xs_ref[...].astype(jnp.bfloat16)
        w1b = w1_ref[0].astype(jnp.bfloat16)
        w3b = w3_ref[0].astype(jnp.bfloat16)
        w2b = w2_ref[0].astype(jnp.bfloat16)

        a = jax.lax.dot_general(xb, w1b, (((1,), (1,)), ((), ())),
                                preferred_element_type=jnp.float32)
        b = jax.lax.dot_general(xb, w3b, (((1,), (1,)), ((), ())),
                                preferred_element_type=jnp.float32)
        h = (a * jax.nn.sigmoid(a) * b).astype(jnp.bfloat16)
        ys_ref[...] = jax.lax.dot_general(h, w2b, (((1,), (1,)), ((), ())),
                                          preferred_element_type=jnp.float32)


def _combine_body(ysg_ref, rw_ref, out_ref):
    rwb = rw_ref[...]
    out_ref[...] = (rwb[:, 0:1] * ysg_ref[:, 0, :]
                    + rwb[:, 1:2] * ysg_ref[:, 1, :])


@jax.jit
def kernel(x, gate_w, w1, w2, w3):
    rw, ps, e_of, used = pl.pallas_call(
        _route_pos_body,
        grid=(2 * NCH,),
        in_specs=[
            pl.BlockSpec((TB, H),
                         lambda g: (jnp.where(g < NCH, g, g - NCH), 0)),
            pl.BlockSpec((E, H), lambda g: (0, 0)),
        ],
        out_specs=[
            pl.BlockSpec((TB, TOP_K),
                         lambda g: (jnp.where(g < NCH, 0, g - NCH), 0)),
            pl.BlockSpec((TB, TOP_K),
                         lambda g: (jnp.where(g < NCH, 0, g - NCH), 0)),
            pl.BlockSpec((NPT, 1), lambda g: (0, 0)),
            pl.BlockSpec((1, 1), lambda g: (0, 0)),
        ],
        out_shape=[
            jax.ShapeDtypeStruct((T, TOP_K), jnp.float32),
            jax.ShapeDtypeStruct((T, TOP_K), jnp.int32),
            jax.ShapeDtypeStruct((NPT, 1), jnp.int32),
            jax.ShapeDtypeStruct((1, 1), jnp.int32),
        ],
        scratch_shapes=[
            pltpu.VMEM((1, E), jnp.float32),
            pltpu.VMEM((1, E), jnp.float32),
            pltpu.VMEM((NCH, TB, TOP_K), jnp.int32),
            pltpu.VMEM((NCH, TB, TOP_K), jnp.float32),
        ],
    )(x, gate_w)
    e_of = e_of.reshape(NPT)
    used = used.reshape(1)

    ps0 = ps[:, 0].reshape(NW, TPW // CCH, CCH)
    ps1 = ps[:, 1].reshape(NW, TPW // CCH, CCH)

    xs = _sc_dispatch(x, ps0, ps1)

    ys = pl.pallas_call(
        _expert_body,
        grid_spec=pltpu.PrefetchScalarGridSpec(
            num_scalar_prefetch=2,
            grid=(NPT,),
            in_specs=[
                pl.BlockSpec((M, H), lambda g, eo, u: (g, 0)),
                pl.BlockSpec((1, I, H), lambda g, eo, u: (eo[g], 0, 0)),
                pl.BlockSpec((1, I, H), lambda g, eo, u: (eo[g], 0, 0)),
                pl.BlockSpec((1, H, I), lambda g, eo, u: (eo[g], 0, 0)),
            ],
            out_specs=pl.BlockSpec((M, H), lambda g, eo, u: (g, 0)),
        ),
        out_shape=jax.ShapeDtypeStruct((PAD, H), jnp.float32),
    )(e_of, used, xs, w1, w3, w2)

    ysg = _sc_combine(ps.reshape(TK), ys).reshape(T, TOP_K, H)

    out = pl.pallas_call(
        _combine_body,
        grid=(NCH,),
        in_specs=[
            pl.BlockSpec((TB, TOP_K, H), lambda t: (t, 0, 0)),
            pl.BlockSpec((TB, TOP_K), lambda t: (t, 0)),
        ],
        out_specs=pl.BlockSpec((TB, H), lambda t: (t, 0)),
        out_shape=jax.ShapeDtypeStruct((T, H), jnp.float32),
    )(ysg, rw)
    return out
